# baseline (speedup 1.0000x reference)
import functools

import jax
import jax.numpy as jnp
from jax import lax
from jax.experimental import pallas as pl
from jax.experimental.pallas import tpu as pltpu
from jax.experimental.pallas import tpu_sc as plsc

_NC = 2
_NS = 16
_NW = _NC * _NS
_GRP = 128
_NBUF = 4
_NPASS = 1
_NSPLIT = 2


@functools.partial(jax.jit, static_argnums=(4, 5))
def _sc_pool(emb, idx2d, tgt3d, zrows, B, DIM):
    groups_w = idx2d.shape[0] // _NW
    bags_w = B // _NW
    gpp = groups_w // _NPASS
    bpp = bags_w // _NPASS
    mesh = plsc.VectorSubcoreMesh(core_axis_name="c", subcore_axis_name="s")

    @functools.partial(
        pl.kernel,
        mesh=mesh,
        out_type=jax.ShapeDtypeStruct((B, DIM), jnp.float32),
        scratch_types=[
            pltpu.VMEM((groups_w, _GRP), jnp.int32),
            pltpu.VMEM((groups_w, _GRP), jnp.int32),
            pltpu.VMEM((_NBUF, _GRP, DIM), jnp.float32),
            pltpu.VMEM_SHARED((_NS * bpp, DIM), jnp.float32),
            pltpu.SemaphoreType.DMA((_NBUF,)),
            pltpu.SemaphoreType.DMA((_NBUF,)),
        ],
    )
    def pool(emb_hbm, idx_hbm, tgt_hbm, zrows_hbm, out_hbm,
             idx_v, tgt_v, rows_v, shared_v, gsem, ssem):
        c = lax.axis_index("c")
        s = lax.axis_index("s")
        wid = c * _NS + s

        pltpu.async_copy(zrows_hbm, shared_v.at[pl.ds(s * bpp, bpp)],
                         ssem.at[0])
        pltpu.sync_copy(idx_hbm.at[pl.ds(wid * groups_w, groups_w)], idx_v)
        pltpu.sync_copy(tgt_hbm.at[s], tgt_v)
        pltpu.make_async_copy(
            zrows_hbm, shared_v.at[pl.ds(s * bpp, bpp)], ssem.at[0]).wait()

        def start_gather(g, b):
            pltpu.async_copy(emb_hbm.at[idx_v.at[g]], rows_v.at[b], gsem.at[b])

        def wait_gather(g, b):
            pltpu.make_async_copy(
                emb_hbm.at[idx_v.at[g]], rows_v.at[b], gsem.at[b]).wait()

        def start_scatter(g, b):
            pltpu.async_copy(
                rows_v.at[b], shared_v.at[tgt_v.at[g]], ssem.at[b], add=True)

        def wait_scatter(g, b):
            pltpu.make_async_copy(
                rows_v.at[b], shared_v.at[tgt_v.at[g]], ssem.at[b]).wait()

        for b in range(_NBUF):
            start_gather(b, b)

        def ring_span(lo, hi):
            @pl.loop(lo, hi, step=_NBUF)
            def _(g0):
                for b in range(_NBUF):
                    g = g0 + b
                    wait_gather(g, b)
                    start_scatter(g, b)
                for b in range(_NBUF):
                    g = g0 + b
                    nxt = g + _NBUF

                    @pl.when(nxt < groups_w)
                    def _():
                        wait_scatter(g, b)
                        start_gather(nxt, b)

        def copy_out(p):
            pltpu.sync_copy(
                shared_v.at[pl.ds(s * bpp, bpp)],
                out_hbm.at[pl.ds(wid * bags_w + p * bpp, bpp)],
            )

        for p in range(_NPASS):
            ring_span(p * gpp, (p + 1) * gpp)
            if p == _NPASS - 1:
                for b in range(_NBUF):
                    wait_scatter(groups_w - _NBUF + b, b)
                copy_out(p)
            else:
                copy_out(p)
                pltpu.sync_copy(zrows_hbm, shared_v.at[pl.ds(s * bpp, bpp)])

    return pool(emb, idx2d, tgt3d, zrows)


def _tc_project(pooled, length2d, W, b2d):
    B, DIM = pooled.shape
    HID = W.shape[0]
    BLK = 2048

    def body(p_ref, l_ref, w_ref, b_ref, o_ref):
        recip = 1.0 / l_ref[...].astype(jnp.float32)
        s = p_ref[...] * recip
        o_ref[...] = lax.dot_general(
            s, w_ref[...], (((1,), (1,)), ((), ())),
            preferred_element_type=jnp.float32,
        ) + b_ref[...]

    return pl.pallas_call(
        body,
        grid=(B // BLK,),
        in_specs=[
            pl.BlockSpec((BLK, DIM), lambda i: (i, 0)),
            pl.BlockSpec((BLK, 1), lambda i: (i, 0)),
            pl.BlockSpec((HID, DIM), lambda i: (0, 0)),
            pl.BlockSpec((1, HID), lambda i: (0, 0)),
        ],
        out_specs=pl.BlockSpec((BLK, HID), lambda i: (i, 0)),
        out_shape=jax.ShapeDtypeStruct((B, HID), jnp.float32),
    )(pooled, length2d, W, b2d)


def kernel(x, length, emb, W, b):
    B, L = x.shape
    DIM = emb.shape[1]
    HID = W.shape[0]
    Bh = B // _NSPLIT
    bags_w = Bh // _NW
    bpp = bags_w // _NPASS
    span = (Bh * L // _NW) // _NPASS

    idx2d = x.astype(jnp.int32).reshape(B * L // _GRP, _GRP)
    rows_h = idx2d.shape[0] // _NSPLIT
    f = jnp.arange(Bh * L // _NW, dtype=jnp.int32)
    local = (f % span) // L
    tgt3d = (local[None, :] + bpp * jnp.arange(_NS, dtype=jnp.int32)[:, None]
             ).reshape(_NS, -1, _GRP)
    zrows = jnp.zeros((bpp, DIM), jnp.float32)

    length2d = length.reshape(B, 1)
    b2d = b.reshape(1, HID)
    outs = []
    for h in range(_NSPLIT):
        pooled = _sc_pool(emb, idx2d[h * rows_h:(h + 1) * rows_h],
                          tgt3d, zrows, Bh, DIM)
        outs.append(_tc_project(
            pooled, length2d[h * Bh:(h + 1) * Bh], W, b2d))
    return jnp.concatenate(outs, axis=0)


# speedup vs baseline: 1.0651x; 1.0651x over previous
import functools

import jax
import jax.numpy as jnp
from jax import lax
from jax.experimental import pallas as pl
from jax.experimental.pallas import tpu as pltpu
from jax.experimental.pallas import tpu_sc as plsc

_NC = 2
_NS = 16
_NW = _NC * _NS
_GRP = 128
_NBUF = 2
_NPASS = 1
_NSPLIT = 1


@functools.partial(jax.jit, static_argnums=(4, 5))
def _sc_pool(emb, idx2d, tgt3d, zrows, B, DIM):
    groups_w = idx2d.shape[0] // _NW
    bags_w = B // _NW
    gpp = groups_w // _NPASS
    bpp = bags_w // _NPASS
    mesh = plsc.VectorSubcoreMesh(core_axis_name="c", subcore_axis_name="s")

    @functools.partial(
        pl.kernel,
        mesh=mesh,
        out_type=jax.ShapeDtypeStruct((B, DIM), jnp.float32),
        scratch_types=[
            pltpu.VMEM((groups_w, _GRP), jnp.int32),
            pltpu.VMEM((groups_w, _GRP), jnp.int32),
            pltpu.VMEM((_NBUF, _GRP, DIM), jnp.float32),
            pltpu.VMEM_SHARED((_NS * bpp, DIM), jnp.float32),
            pltpu.SemaphoreType.DMA((_NBUF,)),
            pltpu.SemaphoreType.DMA((_NBUF,)),
        ],
    )
    def pool(emb_hbm, idx_hbm, tgt_hbm, zrows_hbm, out_hbm,
             idx_v, tgt_v, rows_v, shared_v, gsem, ssem):
        c = lax.axis_index("c")
        s = lax.axis_index("s")
        wid = c * _NS + s

        pltpu.async_copy(zrows_hbm, shared_v.at[pl.ds(s * bpp, bpp)],
                         ssem.at[0])
        pltpu.sync_copy(idx_hbm.at[pl.ds(wid * groups_w, groups_w)], idx_v)
        pltpu.sync_copy(tgt_hbm.at[s], tgt_v)
        pltpu.make_async_copy(
            zrows_hbm, shared_v.at[pl.ds(s * bpp, bpp)], ssem.at[0]).wait()

        def start_gather(g, b):
            pltpu.async_copy(emb_hbm.at[idx_v.at[g]], rows_v.at[b], gsem.at[b])

        def wait_gather(g, b):
            pltpu.make_async_copy(
                emb_hbm.at[idx_v.at[g]], rows_v.at[b], gsem.at[b]).wait()

        def start_scatter(g, b):
            pltpu.async_copy(
                rows_v.at[b], shared_v.at[tgt_v.at[g]], ssem.at[b], add=True)

        def wait_scatter(g, b):
            pltpu.make_async_copy(
                rows_v.at[b], shared_v.at[tgt_v.at[g]], ssem.at[b]).wait()

        for b in range(_NBUF):
            start_gather(b, b)

        def ring_span(lo, hi):
            @pl.loop(lo, hi, step=_NBUF)
            def _(g0):
                for b in range(_NBUF):
                    g = g0 + b
                    wait_gather(g, b)
                    start_scatter(g, b)
                for b in range(_NBUF):
                    g = g0 + b
                    nxt = g + _NBUF

                    @pl.when(nxt < groups_w)
                    def _():
                        wait_scatter(g, b)
                        start_gather(nxt, b)

        def copy_out(p):
            pltpu.sync_copy(
                shared_v.at[pl.ds(s * bpp, bpp)],
                out_hbm.at[pl.ds(wid * bags_w + p * bpp, bpp)],
            )

        for p in range(_NPASS):
            ring_span(p * gpp, (p + 1) * gpp)
            if p == _NPASS - 1:
                for b in range(_NBUF):
                    wait_scatter(groups_w - _NBUF + b, b)
                copy_out(p)
            else:
                copy_out(p)
                pltpu.sync_copy(zrows_hbm, shared_v.at[pl.ds(s * bpp, bpp)])

    return pool(emb, idx2d, tgt3d, zrows)


def _tc_project(pooled, length2d, W, b2d):
    B, DIM = pooled.shape
    HID = W.shape[0]
    BLK = 2048

    def body(p_ref, l_ref, w_ref, b_ref, o_ref):
        recip = 1.0 / l_ref[...].astype(jnp.float32)
        s = p_ref[...] * recip
        o_ref[...] = lax.dot_general(
            s, w_ref[...], (((1,), (1,)), ((), ())),
            preferred_element_type=jnp.float32,
        ) + b_ref[...]

    return pl.pallas_call(
        body,
        grid=(B // BLK,),
        in_specs=[
            pl.BlockSpec((BLK, DIM), lambda i: (i, 0)),
            pl.BlockSpec((BLK, 1), lambda i: (i, 0)),
            pl.BlockSpec((HID, DIM), lambda i: (0, 0)),
            pl.BlockSpec((1, HID), lambda i: (0, 0)),
        ],
        out_specs=pl.BlockSpec((BLK, HID), lambda i: (i, 0)),
        out_shape=jax.ShapeDtypeStruct((B, HID), jnp.float32),
    )(pooled, length2d, W, b2d)


def kernel(x, length, emb, W, b):
    B, L = x.shape
    DIM = emb.shape[1]
    HID = W.shape[0]
    Bh = B // _NSPLIT
    bags_w = Bh // _NW
    bpp = bags_w // _NPASS
    span = (Bh * L // _NW) // _NPASS

    idx2d = x.astype(jnp.int32).reshape(B * L // _GRP, _GRP)
    rows_h = idx2d.shape[0] // _NSPLIT
    f = jnp.arange(Bh * L // _NW, dtype=jnp.int32)
    local = (f % span) // L
    tgt3d = (local[None, :] + bpp * jnp.arange(_NS, dtype=jnp.int32)[:, None]
             ).reshape(_NS, -1, _GRP)
    zrows = jnp.zeros((bpp, DIM), jnp.float32)

    length2d = length.reshape(B, 1)
    b2d = b.reshape(1, HID)
    outs = []
    for h in range(_NSPLIT):
        pooled = _sc_pool(emb, idx2d[h * rows_h:(h + 1) * rows_h],
                          tgt3d, zrows, Bh, DIM)
        outs.append(_tc_project(
            pooled, length2d[h * Bh:(h + 1) * Bh], W, b2d))
    return jnp.concatenate(outs, axis=0)


# speedup vs baseline: 1.2931x; 1.2141x over previous
import functools

import jax
import jax.numpy as jnp
from jax import lax
from jax.experimental import pallas as pl
from jax.experimental.pallas import tpu as pltpu
from jax.experimental.pallas import tpu_sc as plsc

_NC = 2
_NS = 16
_NW = _NC * _NS
_GRP = 128
_NBUF = 4
_NPASS = 2
_NSPLIT = 1


@functools.partial(jax.jit, static_argnums=(4, 5))
def _sc_pool(emb, idx2d, tgt3d, zrows, B, DIM):
    groups_w = idx2d.shape[0] // _NW
    bags_w = B // _NW
    gpp = groups_w // _NPASS
    bpp = bags_w // _NPASS
    mesh = plsc.VectorSubcoreMesh(core_axis_name="c", subcore_axis_name="s")

    @functools.partial(
        pl.kernel,
        mesh=mesh,
        out_type=jax.ShapeDtypeStruct((B, DIM), jnp.float32),
        scratch_types=[
            pltpu.VMEM((groups_w, _GRP), jnp.int32),
            pltpu.VMEM((groups_w, _GRP), jnp.int32),
            pltpu.VMEM((_NBUF, _GRP, DIM), jnp.float32),
            pltpu.VMEM_SHARED((_NS * bpp, DIM), jnp.float32),
            pltpu.SemaphoreType.DMA((_NBUF,)),
            pltpu.SemaphoreType.DMA((_NBUF,)),
        ],
    )
    def pool(emb_hbm, idx_hbm, tgt_hbm, zrows_hbm, out_hbm,
             idx_v, tgt_v, rows_v, shared_v, gsem, ssem):
        c = lax.axis_index("c")
        s = lax.axis_index("s")
        wid = c * _NS + s

        pltpu.async_copy(zrows_hbm, shared_v.at[pl.ds(s * bpp, bpp)],
                         ssem.at[0])
        pltpu.sync_copy(idx_hbm.at[pl.ds(wid * groups_w, groups_w)], idx_v)
        pltpu.sync_copy(tgt_hbm.at[s], tgt_v)
        pltpu.make_async_copy(
            zrows_hbm, shared_v.at[pl.ds(s * bpp, bpp)], ssem.at[0]).wait()

        def start_gather(g, b):
            pltpu.async_copy(emb_hbm.at[idx_v.at[g]], rows_v.at[b], gsem.at[b])

        def wait_gather(g, b):
            pltpu.make_async_copy(
                emb_hbm.at[idx_v.at[g]], rows_v.at[b], gsem.at[b]).wait()

        def start_scatter(g, b):
            pltpu.async_copy(
                rows_v.at[b], shared_v.at[tgt_v.at[g]], ssem.at[b], add=True)

        def wait_scatter(g, b):
            pltpu.make_async_copy(
                rows_v.at[b], shared_v.at[tgt_v.at[g]], ssem.at[b]).wait()

        for b in range(_NBUF):
            start_gather(b, b)

        def ring_span(lo, hi):
            @pl.loop(lo, hi, step=_NBUF)
            def _(g0):
                for b in range(_NBUF):
                    g = g0 + b
                    wait_gather(g, b)
                    start_scatter(g, b)
                    wait_scatter(g, b)
                    nxt = g + _NBUF

                    @pl.when(nxt < groups_w)
                    def _():
                        start_gather(nxt, b)

        def copy_out(p):
            pltpu.sync_copy(
                shared_v.at[pl.ds(s * bpp, bpp)],
                out_hbm.at[pl.ds(wid * bags_w + p * bpp, bpp)],
            )

        for p in range(_NPASS):
            ring_span(p * gpp, (p + 1) * gpp)
            if p == _NPASS - 1:
                copy_out(p)
            else:
                copy_out(p)
                pltpu.sync_copy(zrows_hbm, shared_v.at[pl.ds(s * bpp, bpp)])

    return pool(emb, idx2d, tgt3d, zrows)


def _tc_project(pooled, length2d, W, b2d):
    B, DIM = pooled.shape
    HID = W.shape[0]
    BLK = 2048

    def body(p_ref, l_ref, w_ref, b_ref, o_ref):
        recip = 1.0 / l_ref[...].astype(jnp.float32)
        s = p_ref[...] * recip
        o_ref[...] = lax.dot_general(
            s, w_ref[...], (((1,), (1,)), ((), ())),
            preferred_element_type=jnp.float32,
        ) + b_ref[...]

    return pl.pallas_call(
        body,
        grid=(B // BLK,),
        in_specs=[
            pl.BlockSpec((BLK, DIM), lambda i: (i, 0)),
            pl.BlockSpec((BLK, 1), lambda i: (i, 0)),
            pl.BlockSpec((HID, DIM), lambda i: (0, 0)),
            pl.BlockSpec((1, HID), lambda i: (0, 0)),
        ],
        out_specs=pl.BlockSpec((BLK, HID), lambda i: (i, 0)),
        out_shape=jax.ShapeDtypeStruct((B, HID), jnp.float32),
    )(pooled, length2d, W, b2d)


def kernel(x, length, emb, W, b):
    B, L = x.shape
    DIM = emb.shape[1]
    HID = W.shape[0]
    Bh = B // _NSPLIT
    bags_w = Bh // _NW
    bpp = bags_w // _NPASS
    span = (Bh * L // _NW) // _NPASS

    idx2d = x.astype(jnp.int32).reshape(B * L // _GRP, _GRP)
    rows_h = idx2d.shape[0] // _NSPLIT
    f = jnp.arange(Bh * L // _NW, dtype=jnp.int32)
    local = (f % span) // L
    tgt3d = (local[None, :] + bpp * jnp.arange(_NS, dtype=jnp.int32)[:, None]
             ).reshape(_NS, -1, _GRP)
    zrows = jnp.zeros((bpp, DIM), jnp.float32)

    length2d = length.reshape(B, 1)
    b2d = b.reshape(1, HID)
    outs = []
    for h in range(_NSPLIT):
        pooled = _sc_pool(emb, idx2d[h * rows_h:(h + 1) * rows_h],
                          tgt3d, zrows, Bh, DIM)
        outs.append(_tc_project(
            pooled, length2d[h * Bh:(h + 1) * Bh], W, b2d))
    return jnp.concatenate(outs, axis=0)
